# Initial kernel scaffold; baseline (speedup 1.0000x reference)
import jax
import jax.numpy as jnp
from jax.experimental import pallas as pl


def kernel(x, edge_index, edge_attr, batch, W1, b1, W2, b2, Wfc, bfc):
    raise NotImplementedError("write your pallas kernel here")


# speedup vs baseline: 20.0694x; 20.0694x over previous
import functools

import jax
import jax.numpy as jnp
from jax import lax
from jax.experimental import pallas as pl
from jax.experimental.pallas import tpu as pltpu
from jax.experimental.pallas import tpu_sc as plsc

N = 10000
D = 128
G = 64
NP = 10240
R = 640
NBLK = NP // R
NC, NS = 2, 16
RPT = NP // NS
CH = 128
EP = NC * NS * 80 * CH
PER_TILE = EP // (NC * NS)
NCHUNK = PER_TILE // CH
DEGW = 128

_f32 = jnp.float32


@functools.cache
def _mesh():
    return plsc.VectorSubcoreMesh(core_axis_name="c", subcore_axis_name="s",
                                  num_cores=NC, num_subcores=NS)


def _deg_call(dst_pad):
    def body(dst_hbm, out_hbm, idx_v, ones_v, zb, acc):
        cid = lax.axis_index("c")
        sid = lax.axis_index("s")
        for r in range(CH):
            for c in range(DEGW // 16):
                ones_v[r, c * 16:(c + 1) * 16] = jnp.ones((16,), _f32)
        for r in range(16):
            for c in range(DEGW // 16):
                zb[r, c * 16:(c + 1) * 16] = jnp.zeros((16,), _f32)
        rows0 = sid * RPT
        for j in range(RPT // 16):
            pltpu.sync_copy(zb, acc.at[pl.ds(rows0 + j * 16, 16)])
        plsc.subcore_barrier()
        tbase = (cid * NS + sid) * PER_TILE

        def step(t, c):
            pltpu.sync_copy(dst_hbm.at[pl.ds(tbase + t * CH, CH)], idx_v)
            pltpu.sync_copy(ones_v, acc.at[idx_v], add=True)
            return c

        lax.fori_loop(0, NCHUNK, step, 0)
        plsc.subcore_barrier()
        pltpu.sync_copy(acc.at[pl.ds(rows0, RPT)],
                        out_hbm.at[pl.ds(cid * NP + rows0, RPT)])

    return pl.kernel(
        body,
        out_type=jax.ShapeDtypeStruct((NC * NP, DEGW), _f32),
        mesh=_mesh(),
        scratch_types=[
            pltpu.VMEM((CH,), jnp.int32),
            pltpu.VMEM((CH, DEGW), _f32),
            pltpu.VMEM((16, DEGW), _f32),
            pltpu.VMEM_SHARED((NP, DEGW), _f32),
        ],
    )(dst_pad)


def _agg_call(src_pad, dst_pad, hp):
    def body(src_hbm, dst_hbm, hp_hbm, out_hbm,
             sb0, db0, sb1, db1, rb0, rb1, zb, acc, sem0, sem1):
        cid = lax.axis_index("c")
        sid = lax.axis_index("s")
        for r in range(16):
            for c in range(8):
                zb[r, c * 16:(c + 1) * 16] = jnp.zeros((16,), _f32)
        rows0 = sid * RPT
        for j in range(RPT // 16):
            pltpu.sync_copy(zb, acc.at[pl.ds(rows0 + j * 16, 16)])
        plsc.subcore_barrier()
        tbase = (cid * NS + sid) * PER_TILE

        def pair(p, c):
            t0 = tbase + (2 * p) * CH
            t1 = t0 + CH
            pltpu.sync_copy(src_hbm.at[pl.ds(t0, CH)], sb0)
            cp0 = pltpu.async_copy(hp_hbm.at[sb0], rb0, sem0)
            pltpu.sync_copy(src_hbm.at[pl.ds(t1, CH)], sb1)
            cp1 = pltpu.async_copy(hp_hbm.at[sb1], rb1, sem1)
            pltpu.sync_copy(dst_hbm.at[pl.ds(t0, CH)], db0)
            pltpu.sync_copy(dst_hbm.at[pl.ds(t1, CH)], db1)
            cp0.wait()
            pltpu.sync_copy(rb0, acc.at[db0], add=True)
            cp1.wait()
            pltpu.sync_copy(rb1, acc.at[db1], add=True)
            return c

        lax.fori_loop(0, NCHUNK // 2, pair, 0)
        plsc.subcore_barrier()
        pltpu.sync_copy(acc.at[pl.ds(rows0, RPT)],
                        out_hbm.at[pl.ds(cid * NP + rows0, RPT)])

    return pl.kernel(
        body,
        out_type=jax.ShapeDtypeStruct((NC * NP, D), _f32),
        mesh=_mesh(),
        scratch_types=[
            pltpu.VMEM((CH,), jnp.int32),
            pltpu.VMEM((CH,), jnp.int32),
            pltpu.VMEM((CH,), jnp.int32),
            pltpu.VMEM((CH,), jnp.int32),
            pltpu.VMEM((CH, D), _f32),
            pltpu.VMEM((CH, D), _f32),
            pltpu.VMEM((16, D), _f32),
            pltpu.VMEM_SHARED((NP, D), _f32),
            pltpu.SemaphoreType.DMA,
            pltpu.SemaphoreType.DMA,
        ],
    )(src_pad, dst_pad, hp)


def _prep_body(x_ref, w_ref, d0_ref, d1_ref, o_ref):
    dinv = lax.rsqrt(d0_ref[:, 0] + d1_ref[:, 0] + 1.0)
    o_ref[...] = jnp.dot(x_ref[...], w_ref[...],
                         preferred_element_type=_f32) * dinv[:, None]


def _mid_body(p0_ref, p1_ref, hp_ref, d0_ref, d1_ref, b1_ref, w2_ref, o_ref):
    dinv = lax.rsqrt(d0_ref[:, 0] + d1_ref[:, 0] + 1.0)
    y = (p0_ref[...] + p1_ref[...] + hp_ref[...]) * dinv[:, None] + b1_ref[...]
    y = jnp.maximum(y, 0.0)
    o_ref[...] = jnp.dot(y, w2_ref[...],
                         preferred_element_type=_f32) * dinv[:, None]


def _final_body(q0_ref, q1_ref, hp_ref, d0_ref, d1_ref, b2_ref, bt_ref,
                wfc_ref, bfc_ref, o_ref, psum, csum):
    i = pl.program_id(0)

    @pl.when(i == 0)
    def _():
        psum[...] = jnp.zeros_like(psum)
        csum[...] = jnp.zeros_like(csum)

    dinv = lax.rsqrt(d0_ref[:, 0] + d1_ref[:, 0] + 1.0)
    h = (q0_ref[...] + q1_ref[...] + hp_ref[...]) * dinv[:, None] + b2_ref[...]
    rows = i * R + lax.broadcasted_iota(jnp.int32, (R, 1), 0)
    valid = rows < N
    h = jnp.where(valid, h, 0.0)
    bt = bt_ref[0, 0, :][:, None]
    gid = lax.broadcasted_iota(jnp.int32, (1, G), 1)
    onehot = (bt == gid).astype(_f32) * valid.astype(_f32)
    psum[...] += lax.dot_general(onehot, h, (((0,), (0,)), ((), ())),
                                 preferred_element_type=_f32)
    csum[...] += jnp.sum(onehot, axis=0)[:, None]
    pooled = psum[...] / jnp.maximum(csum[...], 1.0)
    o_ref[...] = jnp.dot(pooled, wfc_ref[...],
                         preferred_element_type=_f32) + bfc_ref[...]


def _row_spec(off):
    return pl.BlockSpec((R, D), lambda i, off=off: (i + off, 0))


def _deg_spec(off):
    return pl.BlockSpec((R, DEGW), lambda i, off=off: (i + off, 0))


def _full_spec(shape):
    nd = len(shape)
    return pl.BlockSpec(shape, lambda i: (0,) * nd)


def _prep_call(x, W1, deg2):
    return pl.pallas_call(
        _prep_body,
        grid=(NBLK,),
        in_specs=[_row_spec(0), _full_spec((D, D)), _deg_spec(0), _deg_spec(NBLK)],
        out_specs=_row_spec(0),
        out_shape=jax.ShapeDtypeStruct((N, D), _f32),
    )(x, W1, deg2, deg2)


def _mid_call(P, hp1, deg2, b1, W2):
    return pl.pallas_call(
        _mid_body,
        grid=(NBLK,),
        in_specs=[_row_spec(0), _row_spec(NBLK), _row_spec(0),
                  _deg_spec(0), _deg_spec(NBLK),
                  _full_spec((1, D)), _full_spec((D, D))],
        out_specs=_row_spec(0),
        out_shape=jax.ShapeDtypeStruct((N, D), _f32),
    )(P, P, hp1, deg2, deg2, b1, W2)


def _final_call(Q, hp2, deg2, b2, batch3, Wfc, bfc):
    return pl.pallas_call(
        _final_body,
        grid=(NBLK,),
        in_specs=[_row_spec(0), _row_spec(NBLK), _row_spec(0),
                  _deg_spec(0), _deg_spec(NBLK),
                  _full_spec((1, D)),
                  pl.BlockSpec((1, 1, R), lambda i: (i, 0, 0)),
                  _full_spec((D, D)), _full_spec((1, D))],
        out_specs=_full_spec((G, D)),
        out_shape=jax.ShapeDtypeStruct((G, D), _f32),
        scratch_shapes=[pltpu.VMEM((G, D), _f32), pltpu.VMEM((G, D), _f32)],
    )(Q, Q, hp2, deg2, deg2, b2, batch3, Wfc, bfc)


def kernel(x, edge_index, edge_attr, batch, W1, b1, W2, b2, Wfc, bfc):
    E = edge_index.shape[1]
    pad = EP - E
    ar = jnp.arange(pad, dtype=jnp.int32)
    src_pad = jnp.concatenate([edge_index[0], ar % 8192])
    dst_pad = jnp.concatenate([edge_index[1], N + (ar % CH)])
    batch3 = jnp.concatenate(
        [batch, jnp.full((NP - N,), G, jnp.int32)]).reshape(NBLK, 1, R)

    deg2 = _deg_call(dst_pad)
    hp1 = _prep_call(x, W1, deg2)
    P = _agg_call(src_pad, dst_pad, hp1)
    hp2 = _mid_call(P, hp1, deg2, b1.reshape(1, D), W2)
    Q = _agg_call(src_pad, dst_pad, hp2)
    return _final_call(Q, hp2, deg2, b2.reshape(1, D), batch3, Wfc,
                       bfc.reshape(1, D))
